# baseline (speedup 1.0000x reference)
import functools

import jax
import jax.numpy as jnp
from jax import lax
from jax.experimental import pallas as pl
from jax.experimental.pallas import tpu as pltpu
from jax.experimental.pallas import tpu_sc as plsc

N = 10000
NPAD = 10240
E = 320000
D = 128
C = 8
B = 16
K = 496
KP = 512
NT = 16
NC = 2
ROWS_T = NPAD // NT
CHUNK = 128
NCH = 160
NSLOT = 4
EPT = NCH * CHUNK
L_ITERS = 10
STEPS = 8
NB = 10
BLK = 1024

_f32 = jnp.float32
_i32 = jnp.int32


def _h0_body(x_ref, w_ref, o_ref):
    o_ref[...] = jnp.dot(x_ref[...], w_ref[...], preferred_element_type=_f32)


def _compute_h0(xp, w_inp):
    return pl.pallas_call(
        _h0_body,
        grid=(NB,),
        in_specs=[
            pl.BlockSpec((BLK, D), lambda i: (i, 0)),
            pl.BlockSpec((D, D), lambda i: (0, 0)),
        ],
        out_specs=pl.BlockSpec((BLK, D), lambda i: (i, 0)),
        out_shape=jax.ShapeDtypeStruct((NPAD, D), _f32),
    )(xp, w_inp)


def _sc_tanh(x):
    return 1.0 - 2.0 / (jnp.exp(2.0 * x) + 1.0)


def _sc_round_bf16(x):
    u = lax.bitcast_convert_type(x, jnp.int32)
    r = u + 0x7FFF + ((u >> 16) & 1)
    r = r & jnp.int32(-65536)
    return lax.bitcast_convert_type(r, _f32)


_GATHER_DNUMS = lax.GatherDimensionNumbers(
    offset_dims=(), collapsed_slice_dims=(0,), start_index_map=(0,))


def _sc_body(h0_ref, gidx_ref, sidx_ref, arep_ref, z_ref, hall_ref,
             acc, gi, si, rows4, h0s, hw, hr, accv, zbuf, arep,
             g0, g1, g2, g3, s0, s1, s2, s3):
    gsem = [g0, g1, g2, g3]
    ssem = [s0, s1, s2, s3]
    cid = lax.axis_index("c")
    sid = lax.axis_index("s")
    r0 = sid * ROWS_T
    hbase = cid * NPAD + r0

    pltpu.sync_copy(gidx_ref.at[cid, sid], gi)
    pltpu.sync_copy(sidx_ref.at[cid, sid], si)
    pltpu.sync_copy(h0_ref.at[pl.ds(r0, ROWS_T)], h0s)
    pltpu.sync_copy(z_ref, zbuf)
    pltpu.sync_copy(arep_ref.at[cid], arep)

    def init_round(p, c2):
        hr[p, :] = _sc_round_bf16(h0s[p, :])
        return c2

    lax.fori_loop(0, ROWS_T, init_round, 0)
    pltpu.sync_copy(hr, hall_ref.at[pl.ds(hbase, ROWS_T)])
    plsc.subcore_barrier()

    splat_idx = [jnp.full((16, 1), k, _i32) for k in range(C)]

    def iteration(_, carry):
        for u in range(NSLOT):
            pltpu.async_copy(hall_ref.at[gi.at[u]], rows4.at[u], gsem[u])
        pltpu.sync_copy(zbuf, acc.at[pl.ds(r0, ROWS_T)])
        plsc.subcore_barrier()

        def chunk(t, c2):
            j = t * NSLOT
            for u in range(NSLOT):
                pltpu.make_async_copy(hall_ref.at[gi.at[u]],
                                      rows4.at[u], gsem[u]).wait()
                pltpu.async_copy(rows4.at[u], acc.at[si.at[j + u]],
                                 ssem[u], add=True)
            for u in range(NSLOT):
                pltpu.make_async_copy(rows4.at[u],
                                      acc.at[si.at[0]], ssem[u]).wait()
                pltpu.async_copy(hall_ref.at[gi.at[j + NSLOT + u]],
                                 rows4.at[u], gsem[u])
            return c2

        lax.fori_loop(0, NCH // NSLOT, chunk, 0)
        for u in range(NSLOT):
            pltpu.make_async_copy(hall_ref.at[gi.at[u]],
                                  rows4.at[u], gsem[u]).wait()
        plsc.subcore_barrier()

        pltpu.sync_copy(acc.at[pl.ds(r0, ROWS_T)], accv)

        def row_upd(p4, c2):
            for u in range(4):
                p = p4 * 4 + u
                v = accv[p, :]
                o = h0s[p, :]
                for k in range(C):
                    sk = lax.gather(
                        v, splat_idx[k], _GATHER_DNUMS, slice_sizes=(1,),
                        mode=lax.GatherScatterMode.PROMISE_IN_BOUNDS)
                    o = o + sk * arep[k, :]
                t = _sc_tanh(o)
                hw[p, :] = t
                hr[p, :] = _sc_round_bf16(t)
            return c2

        lax.fori_loop(0, ROWS_T // 4, row_upd, 0)
        pltpu.sync_copy(hr, hall_ref.at[pl.ds(hbase, ROWS_T)])
        plsc.subcore_barrier()
        return carry

    lax.fori_loop(0, L_ITERS, iteration, 0)
    pltpu.sync_copy(hw, hall_ref.at[pl.ds(hbase, ROWS_T)])


def _run_sc(h0, gidx, sidx, a2, zeros):
    mesh = plsc.VectorSubcoreMesh(core_axis_name="c", subcore_axis_name="s")
    fn = pl.kernel(
        _sc_body,
        out_type=jax.ShapeDtypeStruct((NC * NPAD, 16), _f32),
        mesh=mesh,
        scratch_types=[
            pltpu.VMEM_SHARED((NPAD, 16), _f32),
            pltpu.VMEM((NCH + NSLOT, CHUNK), _i32),
            pltpu.VMEM((NCH, CHUNK), _i32),
            pltpu.VMEM((NSLOT, CHUNK, 16), _f32),
            pltpu.VMEM((ROWS_T, 16), _f32),
            pltpu.VMEM((ROWS_T, 16), _f32),
            pltpu.VMEM((ROWS_T, 16), _f32),
            pltpu.VMEM((ROWS_T, 16), _f32),
            pltpu.VMEM((ROWS_T, 16), _f32),
            pltpu.VMEM((C, 16), _f32),
            pltpu.SemaphoreType.DMA,
            pltpu.SemaphoreType.DMA,
            pltpu.SemaphoreType.DMA,
            pltpu.SemaphoreType.DMA,
            pltpu.SemaphoreType.DMA,
            pltpu.SemaphoreType.DMA,
            pltpu.SemaphoreType.DMA,
            pltpu.SemaphoreType.DMA,
        ],
        compiler_params=pltpu.CompilerParams(use_tc_tiling_on_sc=False),
    )
    return fn(h0, gidx, sidx, a2, zeros)


def _blk(b):
    return pl.ds(b * BLK, BLK)


def _bmask(b):
    return (lax.broadcasted_iota(_i32, (BLK, 1), 0) + b * BLK < N).astype(_f32)


_DN0 = (((0,), (0,)), ((), ()))


def _final_body(hcat_ref, w2_ref, contr_ref, oh_ref, wxq_ref, wxr_ref, whp_ref,
                bl_ref, woq_ref, wor_ref, bo_ref,
                out_ref, pg_ref, c_scr, neg_scr, p_scr):
    def p1(b, s1):
        neg = jax.nn.softplus(jnp.dot(hcat_ref[_blk(b), :], w2_ref[...],
                                      preferred_element_type=_f32)) * _bmask(b)
        neg_scr[_blk(b), :] = neg
        return s1 + jnp.sum(neg, axis=0, keepdims=True)

    s1 = lax.fori_loop(0, NB, p1, jnp.zeros((1, 32), _f32))
    mean = s1 * (1.0 / N)

    def p1b(b, s2):
        d = (neg_scr[_blk(b), :] - mean) * _bmask(b)
        return s2 + jnp.sum(d * d, axis=0, keepdims=True)

    s2 = lax.fori_loop(0, NB, p1b, jnp.zeros((1, 32), _f32))
    inv = lax.rsqrt(s2 * (1.0 / N) + 1e-5)

    def p2(b, carry):
        seg, cnt = carry
        negb = neg_scr[_blk(b), :]
        c_scr[_blk(b), :] = jnp.tanh(
            jnp.dot(((negb - mean) * inv).astype(jnp.bfloat16), contr_ref[...],
                    preferred_element_type=_f32))
        ohb = oh_ref[_blk(b), :]
        seg = seg + lax.dot_general(ohb, negb, _DN0, preferred_element_type=_f32)
        cnt = cnt + lax.dot_general(ohb, _bmask(b), _DN0,
                                    preferred_element_type=_f32)
        return seg, cnt

    seg, cnt = lax.fori_loop(
        0, NB, p2, (jnp.zeros((B, 32), _f32), jnp.zeros((B, 1), _f32)))
    pg = seg / cnt
    pg_ref[...] = jnp.broadcast_to(jnp.sum(pg, 0, keepdims=True) * (1.0 / B),
                                   (8, 32))

    q = jnp.zeros((B, KP), _f32)
    cell = jnp.zeros((B, KP), _f32)
    r = jnp.zeros((B, KP), _f32)
    for _ in range(STEPS):
        qb = q.astype(jnp.bfloat16)
        rb = r.astype(jnp.bfloat16)
        gates = (jnp.dot(qb, wxq_ref[...], preferred_element_type=_f32)
                 + jnp.dot(rb, wxr_ref[...], preferred_element_type=_f32)
                 + jnp.dot(qb, whp_ref[...], preferred_element_type=_f32)
                 + bl_ref[...])
        i_g = gates[:, 0:KP]
        f_g = gates[:, KP:2 * KP]
        g_g = gates[:, 2 * KP:3 * KP]
        o_g = gates[:, 3 * KP:4 * KP]
        cell = jax.nn.sigmoid(f_g) * cell + jax.nn.sigmoid(i_g) * jnp.tanh(g_g)
        q = jax.nn.sigmoid(o_g) * jnp.tanh(cell)

        def pa(b, m):
            pb = lax.dot_general(c_scr[_blk(b), :], q, (((1,), (1,)), ((), ())),
                                 preferred_element_type=_f32)
            p_scr[_blk(b), :] = pb
            mb = jnp.max(jnp.where(oh_ref[_blk(b), :] > 0, pb, -jnp.inf),
                         axis=0, keepdims=True)
            return jnp.maximum(m, mb)

        em = lax.fori_loop(0, NB, pa, jnp.full((1, B), -jnp.inf, _f32))
        em = jnp.where(jnp.isfinite(em), em, 0.0)

        def pb_(b, den):
            ex = oh_ref[_blk(b), :] * jnp.exp(p_scr[_blk(b), :] - em)
            p_scr[_blk(b), :] = ex
            return den + jnp.sum(ex, axis=0, keepdims=True)

        den = lax.fori_loop(0, NB, pb_, jnp.zeros((1, B), _f32))
        inv_den = 1.0 / jnp.where(den > 0, den, 1.0)

        def pc(b, racc):
            wn = p_scr[_blk(b), :] * inv_den
            return racc + lax.dot_general(wn, c_scr[_blk(b), :], _DN0,
                                          preferred_element_type=_f32)

        r = lax.fori_loop(0, NB, pc, jnp.zeros((B, KP), _f32))

    out_ref[...] = (jnp.dot(q.astype(jnp.bfloat16), woq_ref[...],
                            preferred_element_type=_f32)
                    + jnp.dot(r.astype(jnp.bfloat16), wor_ref[...],
                              preferred_element_type=_f32)
                    + bo_ref[...])


def _run_final(hcat, w2, contr_p, oh, wxq, wxr, whp, blp, woq, wor, bop):
    return pl.pallas_call(
        _final_body,
        out_shape=(
            jax.ShapeDtypeStruct((B, 128), _f32),
            jax.ShapeDtypeStruct((8, 32), _f32),
        ),
        scratch_shapes=[
            pltpu.VMEM((NPAD, KP), _f32),
            pltpu.VMEM((NPAD, 32), _f32),
            pltpu.VMEM((NPAD, B), _f32),
        ],
    )(hcat, w2, contr_p, oh, wxq, wxr, whp, blp, woq, wor, bop)


def _gatepad(w):
    w4 = w.reshape(w.shape[0], 4, K)
    return jnp.pad(w4, ((0, 0), (0, 0), (0, KP - K))).reshape(w.shape[0], 4 * KP)


def kernel(x, edge_index, batch_idx, W_in, A_bu, A_td, W_bu, W_td,
           contrastive, Wx, Wh, b_lstm, W_out, b_out):
    x = x.astype(_f32)
    src = edge_index[0].astype(_i32)
    dst = edge_index[1].astype(_i32)
    bidx = batch_idx.astype(_i32)

    _bf = jnp.bfloat16
    xp = jnp.pad(x, ((0, NPAD - N), (0, 0))).astype(_bf)
    w_inp = jnp.pad(W_in, ((0, 0), (0, D - C))).astype(_bf)
    h0 = _compute_h0(xp, w_inp)[:, :16]

    pad = jnp.full((NT * EPT - E,), NPAD - 1, _i32)

    def lay(a):
        return jnp.concatenate([a, pad]).reshape(NT, NCH, CHUNK)

    src_l, dst_l = lay(src), lay(dst)
    gidx = jnp.stack([src_l, dst_l + NPAD])
    gidx = jnp.pad(gidx, ((0, 0), (0, 0), (0, NSLOT), (0, 0)))
    sidx = jnp.stack([dst_l, src_l])
    a2rep = jnp.stack([jnp.pad(A_bu, ((0, 0), (0, C))),
                       jnp.pad(A_td, ((0, 0), (0, C)))])
    a2rep = a2rep.astype(jnp.bfloat16).astype(_f32)
    zeros = jnp.zeros((ROWS_T, 16), _f32)

    hall = _run_sc(h0, gidx, sidx, a2rep, zeros)
    hcat = jnp.concatenate([hall[NPAD:, :C], hall[:NPAD, :C]], axis=1)

    w2 = jnp.zeros((2 * C, 32), _f32)
    w2 = w2.at[:C, :B].set(W_td).at[C:, B:].set(W_bu)
    oh = (bidx[:, None] == jnp.arange(B, dtype=_i32)[None, :]).astype(_f32)
    oh = jnp.pad(oh, ((0, NPAD - N), (0, 0)))
    contr_p = jnp.pad(contrastive, ((0, 0), (0, KP - K)))
    wxq = jnp.pad(_gatepad(Wx[:K]), ((0, KP - K), (0, 0)))
    wxr = jnp.pad(_gatepad(Wx[K:]), ((0, KP - K), (0, 0)))
    whp = jnp.pad(_gatepad(Wh), ((0, KP - K), (0, 0)))
    blp = _gatepad(b_lstm.reshape(1, 4 * K))
    woq = jnp.pad(W_out[:K], ((0, KP - K), (0, 118)))
    wor = jnp.pad(W_out[K:], ((0, KP - K), (0, 118)))
    bop = jnp.pad(b_out.reshape(1, 10), ((0, 0), (0, 118)))

    out_p, pg = _run_final(hcat.astype(_bf), w2.astype(_bf),
                           contr_p.astype(_bf), oh,
                           wxq.astype(_bf), wxr.astype(_bf), whp.astype(_bf),
                           blp, woq.astype(_bf), wor.astype(_bf), bop)
    return (out_p[:, :10], pg[0])


# speedup vs baseline: 1.0980x; 1.0980x over previous
import functools

import jax
import jax.numpy as jnp
from jax import lax
from jax.experimental import pallas as pl
from jax.experimental.pallas import tpu as pltpu
from jax.experimental.pallas import tpu_sc as plsc

N = 10000
NPAD = 10240
E = 320000
D = 128
C = 8
B = 16
K = 496
KP = 512
NT = 16
NC = 2
ROWS_T = NPAD // NT
CHUNK = 128
NCH = 160
NSLOT = 4
EPT = NCH * CHUNK
L_ITERS = 10
STEPS = 8
NB = 10
BLK = 1024

_f32 = jnp.float32
_i32 = jnp.int32


def _h0_body(x_ref, w_ref, o_ref):
    o_ref[...] = jnp.dot(x_ref[...], w_ref[...], preferred_element_type=_f32)


def _compute_h0(xp, w_inp):
    return pl.pallas_call(
        _h0_body,
        grid=(NB,),
        in_specs=[
            pl.BlockSpec((BLK, D), lambda i: (i, 0)),
            pl.BlockSpec((D, D), lambda i: (0, 0)),
        ],
        out_specs=pl.BlockSpec((BLK, D), lambda i: (i, 0)),
        out_shape=jax.ShapeDtypeStruct((NPAD, D), _f32),
    )(xp, w_inp)


def _sc_tanh(x):
    return 1.0 - 2.0 / (jnp.exp(2.0 * x) + 1.0)


def _sc_round_bf16(x):
    u = lax.bitcast_convert_type(x, jnp.int32)
    r = u + 0x7FFF + ((u >> 16) & 1)
    r = r & jnp.int32(-65536)
    return lax.bitcast_convert_type(r, _f32)


_GATHER_DNUMS = lax.GatherDimensionNumbers(
    offset_dims=(), collapsed_slice_dims=(0,), start_index_map=(0,))


def _sc_body(h0_ref, gidx_ref, sidx_ref, a2_ref, z_ref, hall_ref,
             acc, gi, si, rows4, h0s, hw, hr, accv, zbuf, af, asp,
             g0, g1, g2, g3, s0, s1, s2, s3):
    gsem = [g0, g1, g2, g3]
    ssem = [s0, s1, s2, s3]
    del ssem
    cid = lax.axis_index("c")
    sid = lax.axis_index("s")
    r0 = sid * ROWS_T
    hbase = cid * NPAD + r0

    pltpu.sync_copy(gidx_ref.at[cid, sid], gi)
    pltpu.sync_copy(sidx_ref.at[cid, sid], si)
    pltpu.sync_copy(h0_ref.at[pl.ds(r0, ROWS_T)], h0s)
    pltpu.sync_copy(z_ref, zbuf)
    pltpu.sync_copy(a2_ref.at[cid], af)
    for kc in range(C * C):
        asp[kc, :] = plsc.load_gather(af, [jnp.full((16,), kc, _i32)])

    def init_round(g, c2):
        ridx = g * 16 + lax.iota(_i32, 16)
        for c in range(C):
            colc = jnp.full((16,), c, _i32)
            o = plsc.load_gather(h0s, [ridx, colc])
            plsc.store_scatter(hr, [ridx, colc], _sc_round_bf16(o))
        return c2

    lax.fori_loop(0, ROWS_T // 16, init_round, 0)
    pltpu.sync_copy(hr, hall_ref.at[pl.ds(hbase, ROWS_T)])
    plsc.subcore_barrier()

    def iteration(_, carry):
        for u in range(NSLOT):
            pltpu.async_copy(hall_ref.at[gi.at[u]], rows4.at[u], gsem[u])
        pltpu.sync_copy(zbuf, acc.at[pl.ds(r0, ROWS_T)])
        plsc.subcore_barrier()

        def chunk(t, c2):
            j = t * NSLOT
            for u in range(NSLOT):
                pltpu.make_async_copy(hall_ref.at[gi.at[u]],
                                      rows4.at[u], gsem[u]).wait()
                pltpu.sync_copy(rows4.at[u], acc.at[si.at[j + u]], add=True)
                pltpu.async_copy(hall_ref.at[gi.at[j + NSLOT + u]],
                                 rows4.at[u], gsem[u])
            return c2

        lax.fori_loop(0, NCH // NSLOT, chunk, 0)
        for u in range(NSLOT):
            pltpu.make_async_copy(hall_ref.at[gi.at[u]],
                                  rows4.at[u], gsem[u]).wait()
        plsc.subcore_barrier()

        pltpu.sync_copy(acc.at[pl.ds(r0, ROWS_T)], accv)

        def row_upd(g, c2):
            ridx = g * 16 + lax.iota(_i32, 16)
            acck = [plsc.load_gather(accv, [ridx, jnp.full((16,), k, _i32)])
                    for k in range(C)]
            for c in range(C):
                colc = jnp.full((16,), c, _i32)
                o = plsc.load_gather(h0s, [ridx, colc])
                for k in range(C):
                    o = o + acck[k] * asp[k * C + c, :]
                t = _sc_tanh(o)
                plsc.store_scatter(hw, [ridx, colc], t)
                plsc.store_scatter(hr, [ridx, colc], _sc_round_bf16(t))
            return c2

        lax.fori_loop(0, ROWS_T // 16, row_upd, 0)
        pltpu.sync_copy(hr, hall_ref.at[pl.ds(hbase, ROWS_T)])
        plsc.subcore_barrier()
        return carry

    lax.fori_loop(0, L_ITERS, iteration, 0)
    pltpu.sync_copy(hw, hall_ref.at[pl.ds(hbase, ROWS_T)])


def _run_sc(h0, gidx, sidx, a2, zeros):
    mesh = plsc.VectorSubcoreMesh(core_axis_name="c", subcore_axis_name="s")
    fn = pl.kernel(
        _sc_body,
        out_type=jax.ShapeDtypeStruct((NC * NPAD, C), _f32),
        mesh=mesh,
        scratch_types=[
            pltpu.VMEM_SHARED((NPAD, C), _f32),
            pltpu.VMEM((NCH + NSLOT, CHUNK), _i32),
            pltpu.VMEM((NCH, CHUNK), _i32),
            pltpu.VMEM((NSLOT, CHUNK, C), _f32),
            pltpu.VMEM((ROWS_T, C), _f32),
            pltpu.VMEM((ROWS_T, C), _f32),
            pltpu.VMEM((ROWS_T, C), _f32),
            pltpu.VMEM((ROWS_T, C), _f32),
            pltpu.VMEM((ROWS_T, C), _f32),
            pltpu.VMEM((C * C,), _f32),
            pltpu.VMEM((C * C, 16), _f32),
            pltpu.SemaphoreType.DMA,
            pltpu.SemaphoreType.DMA,
            pltpu.SemaphoreType.DMA,
            pltpu.SemaphoreType.DMA,
            pltpu.SemaphoreType.DMA,
            pltpu.SemaphoreType.DMA,
            pltpu.SemaphoreType.DMA,
            pltpu.SemaphoreType.DMA,
        ],
        compiler_params=pltpu.CompilerParams(use_tc_tiling_on_sc=False,
                                             needs_layout_passes=False),
    )
    return fn(h0, gidx, sidx, a2, zeros)


def _blk(b):
    return pl.ds(b * BLK, BLK)


def _bmask(b):
    return (lax.broadcasted_iota(_i32, (BLK, 1), 0) + b * BLK < N).astype(_f32)


_DN0 = (((0,), (0,)), ((), ()))


def _final_body(hcat_ref, w2_ref, contr_ref, oh_ref, wxq_ref, wxr_ref, whp_ref,
                bl_ref, woq_ref, wor_ref, bo_ref,
                out_ref, pg_ref, c_scr, neg_scr, p_scr):
    def p1(b, s1):
        neg = jax.nn.softplus(jnp.dot(hcat_ref[_blk(b), :], w2_ref[...],
                                      preferred_element_type=_f32)) * _bmask(b)
        neg_scr[_blk(b), :] = neg
        return s1 + jnp.sum(neg, axis=0, keepdims=True)

    s1 = lax.fori_loop(0, NB, p1, jnp.zeros((1, 32), _f32))
    mean = s1 * (1.0 / N)

    def p1b(b, s2):
        d = (neg_scr[_blk(b), :] - mean) * _bmask(b)
        return s2 + jnp.sum(d * d, axis=0, keepdims=True)

    s2 = lax.fori_loop(0, NB, p1b, jnp.zeros((1, 32), _f32))
    inv = lax.rsqrt(s2 * (1.0 / N) + 1e-5)

    def p2(b, carry):
        seg, cnt = carry
        negb = neg_scr[_blk(b), :]
        c_scr[_blk(b), :] = jnp.tanh(
            jnp.dot(((negb - mean) * inv).astype(jnp.bfloat16), contr_ref[...],
                    preferred_element_type=_f32))
        ohb = oh_ref[_blk(b), :]
        seg = seg + lax.dot_general(ohb, negb, _DN0, preferred_element_type=_f32)
        cnt = cnt + lax.dot_general(ohb, _bmask(b), _DN0,
                                    preferred_element_type=_f32)
        return seg, cnt

    seg, cnt = lax.fori_loop(
        0, NB, p2, (jnp.zeros((B, 32), _f32), jnp.zeros((B, 1), _f32)))
    pg = seg / cnt
    pg_ref[...] = jnp.broadcast_to(jnp.sum(pg, 0, keepdims=True) * (1.0 / B),
                                   (8, 32))

    q = jnp.zeros((B, KP), _f32)
    cell = jnp.zeros((B, KP), _f32)
    r = jnp.zeros((B, KP), _f32)
    for _ in range(STEPS):
        qb = q.astype(jnp.bfloat16)
        rb = r.astype(jnp.bfloat16)
        gates = (jnp.dot(qb, wxq_ref[...], preferred_element_type=_f32)
                 + jnp.dot(rb, wxr_ref[...], preferred_element_type=_f32)
                 + jnp.dot(qb, whp_ref[...], preferred_element_type=_f32)
                 + bl_ref[...])
        i_g = gates[:, 0:KP]
        f_g = gates[:, KP:2 * KP]
        g_g = gates[:, 2 * KP:3 * KP]
        o_g = gates[:, 3 * KP:4 * KP]
        cell = jax.nn.sigmoid(f_g) * cell + jax.nn.sigmoid(i_g) * jnp.tanh(g_g)
        q = jax.nn.sigmoid(o_g) * jnp.tanh(cell)

        def pa(b, m):
            pb = lax.dot_general(c_scr[_blk(b), :], q, (((1,), (1,)), ((), ())),
                                 preferred_element_type=_f32)
            p_scr[_blk(b), :] = pb
            mb = jnp.max(jnp.where(oh_ref[_blk(b), :] > 0, pb, -jnp.inf),
                         axis=0, keepdims=True)
            return jnp.maximum(m, mb)

        em = lax.fori_loop(0, NB, pa, jnp.full((1, B), -jnp.inf, _f32))
        em = jnp.where(jnp.isfinite(em), em, 0.0)

        def pb_(b, den):
            ex = oh_ref[_blk(b), :] * jnp.exp(p_scr[_blk(b), :] - em)
            p_scr[_blk(b), :] = ex
            return den + jnp.sum(ex, axis=0, keepdims=True)

        den = lax.fori_loop(0, NB, pb_, jnp.zeros((1, B), _f32))
        inv_den = 1.0 / jnp.where(den > 0, den, 1.0)

        def pc(b, racc):
            wn = p_scr[_blk(b), :] * inv_den
            return racc + lax.dot_general(wn, c_scr[_blk(b), :], _DN0,
                                          preferred_element_type=_f32)

        r = lax.fori_loop(0, NB, pc, jnp.zeros((B, KP), _f32))

    out_ref[...] = (jnp.dot(q.astype(jnp.bfloat16), woq_ref[...],
                            preferred_element_type=_f32)
                    + jnp.dot(r.astype(jnp.bfloat16), wor_ref[...],
                              preferred_element_type=_f32)
                    + bo_ref[...])


def _run_final(hcat, w2, contr_p, oh, wxq, wxr, whp, blp, woq, wor, bop):
    return pl.pallas_call(
        _final_body,
        out_shape=(
            jax.ShapeDtypeStruct((B, 128), _f32),
            jax.ShapeDtypeStruct((8, 32), _f32),
        ),
        scratch_shapes=[
            pltpu.VMEM((NPAD, KP), _f32),
            pltpu.VMEM((NPAD, 32), _f32),
            pltpu.VMEM((NPAD, B), _f32),
        ],
    )(hcat, w2, contr_p, oh, wxq, wxr, whp, blp, woq, wor, bop)


def _gatepad(w):
    w4 = w.reshape(w.shape[0], 4, K)
    return jnp.pad(w4, ((0, 0), (0, 0), (0, KP - K))).reshape(w.shape[0], 4 * KP)


def kernel(x, edge_index, batch_idx, W_in, A_bu, A_td, W_bu, W_td,
           contrastive, Wx, Wh, b_lstm, W_out, b_out):
    x = x.astype(_f32)
    src = edge_index[0].astype(_i32)
    dst = edge_index[1].astype(_i32)
    bidx = batch_idx.astype(_i32)

    _bf = jnp.bfloat16
    xp = jnp.pad(x, ((0, NPAD - N), (0, 0))).astype(_bf)
    w_inp = jnp.pad(W_in, ((0, 0), (0, D - C))).astype(_bf)
    h0 = _compute_h0(xp, w_inp)[:, :C]

    pad = jnp.full((NT * EPT - E,), NPAD - 1, _i32)

    def lay(a):
        return jnp.concatenate([a, pad]).reshape(NT, NCH, CHUNK)

    src_l, dst_l = lay(src), lay(dst)
    gidx = jnp.stack([src_l, dst_l + NPAD])
    gidx = jnp.pad(gidx, ((0, 0), (0, 0), (0, NSLOT), (0, 0)))
    sidx = jnp.stack([dst_l, src_l])
    a2 = jnp.stack([A_bu.reshape(C * C), A_td.reshape(C * C)])
    a2 = a2.astype(jnp.bfloat16).astype(_f32)
    zeros = jnp.zeros((ROWS_T, C), _f32)

    hall = _run_sc(h0, gidx, sidx, a2, zeros)
    hcat = jnp.concatenate([hall[NPAD:], hall[:NPAD]], axis=1)

    w2 = jnp.zeros((2 * C, 32), _f32)
    w2 = w2.at[:C, :B].set(W_td).at[C:, B:].set(W_bu)
    oh = (bidx[:, None] == jnp.arange(B, dtype=_i32)[None, :]).astype(_f32)
    oh = jnp.pad(oh, ((0, NPAD - N), (0, 0)))
    contr_p = jnp.pad(contrastive, ((0, 0), (0, KP - K)))
    wxq = jnp.pad(_gatepad(Wx[:K]), ((0, KP - K), (0, 0)))
    wxr = jnp.pad(_gatepad(Wx[K:]), ((0, KP - K), (0, 0)))
    whp = jnp.pad(_gatepad(Wh), ((0, KP - K), (0, 0)))
    blp = _gatepad(b_lstm.reshape(1, 4 * K))
    woq = jnp.pad(W_out[:K], ((0, KP - K), (0, 118)))
    wor = jnp.pad(W_out[K:], ((0, KP - K), (0, 118)))
    bop = jnp.pad(b_out.reshape(1, 10), ((0, 0), (0, 118)))

    out_p, pg = _run_final(hcat.astype(_bf), w2.astype(_bf),
                           contr_p.astype(_bf), oh,
                           wxq.astype(_bf), wxr.astype(_bf), whp.astype(_bf),
                           blp, woq.astype(_bf), wor.astype(_bf), bop)
    return (out_p[:, :10], pg[0])
